# baseline (speedup 1.0000x reference)
import functools

import jax
import jax.numpy as jnp
from jax import lax
from jax.experimental import pallas as pl
from jax.experimental.pallas import tpu as pltpu
from jax.experimental.pallas import tpu_sc as plsc

DIM = 32
NEG = 5
NCHUNK = 4


def _make_sc_kernel(B):
    info = plsc.get_sparse_core_info()
    NC, NS, L = info.num_cores, info.num_subcores, info.num_lanes
    NW = NC * NS
    assert B % NW == 0 and DIM == 2 * L
    bpw = B // NW
    nbw = bpw * NEG
    ncs = nbw // NCHUNK
    bpc = bpw // NCHUNK

    mesh = plsc.VectorSubcoreMesh(core_axis_name="c", subcore_axis_name="s")

    @functools.partial(
        pl.kernel,
        mesh=mesh,
        out_type=jax.ShapeDtypeStruct((NW, 128), jnp.float32),
        compiler_params=pltpu.CompilerParams(use_tc_tiling_on_sc=True),
        scratch_types=[
            pltpu.VMEM((ncs,), jnp.int32),
            pltpu.VMEM((bpw,), jnp.int32),
            pltpu.VMEM((bpw,), jnp.int32),
            pltpu.VMEM((bpw,), jnp.int32),
            pltpu.VMEM((bpw,), jnp.int32),
            pltpu.VMEM((ncs,), jnp.int32),
            pltpu.VMEM((ncs,), jnp.int32),
            pltpu.VMEM((bpw,), jnp.int32),
            pltpu.VMEM((bpw,), jnp.int32),
            pltpu.VMEM((bpw,), jnp.int32),
            pltpu.VMEM((bpw,), jnp.int32),
            pltpu.VMEM((ncs,), jnp.int32),
            pltpu.VMEM((ncs,), jnp.int32),
            pltpu.VMEM((bpw, 128), jnp.float32),
            pltpu.VMEM((bpw, 128), jnp.float32),
            pltpu.VMEM((bpw, 128), jnp.float32),
            pltpu.VMEM((bpw, 128), jnp.float32),
            pltpu.VMEM((ncs, 128), jnp.float32),
            pltpu.VMEM((ncs, 128), jnp.float32),
            pltpu.VMEM((bpw, DIM), jnp.float32),
            pltpu.VMEM((128,), jnp.float32),
            pltpu.SemaphoreType.DMA,
        ],
    )
    def sc_kernel(rupos_h, aupos_h, rvpos_h, avpos_h, rnpos_h, anpos_h,
                  ruw_h, auw_h, rvw_h, avw_h, out_h,
                  iv, i4ru, i4au, i4rv, i4av, i4n1, i4n2,
                  m4ru, m4au, m4rv, m4av, m4n1, m4n2,
                  bru, bau, brv, bav, bn1, bn2, uu, ostage, sem):
        wid = lax.axis_index("s") * NC + lax.axis_index("c")
        base = wid * bpw
        nbase = wid * nbw

        def stage(idx_h, off, n, i4ref, m4ref):
            pltpu.sync_copy(idx_h.at[pl.ds(off, n)], iv.at[pl.ds(0, n)])

            def sh(t, _):
                raw = iv[pl.ds(16 * t, 16)]
                i4ref[pl.ds(16 * t, 16)] = lax.shift_right_logical(raw, 2)
                m4ref[pl.ds(16 * t, 16)] = (raw & 3) * 32
                return 0
            lax.fori_loop(0, n // 16, sh, 0)

        stage(rupos_h, base, bpw, i4ru, m4ru)
        stage(aupos_h, base, bpw, i4au, m4au)
        stage(rvpos_h, base, bpw, i4rv, m4rv)
        stage(avpos_h, base, bpw, i4av, m4av)

        d0 = pltpu.async_copy(ruw_h.at[i4ru], bru, sem)
        d1 = pltpu.async_copy(auw_h.at[i4au], bau, sem)
        d2 = pltpu.async_copy(rvw_h.at[i4rv], brv, sem)
        d3 = pltpu.async_copy(avw_h.at[i4av], bav, sem)
        d0.wait(); d1.wait(); d2.wait(); d3.wait()

        def pos_body(g, carry):
            s0, s1 = carry
            vru = m4ru[pl.ds(16 * g, 16)]
            vau = m4au[pl.ds(16 * g, 16)]
            vrv = m4rv[pl.ds(16 * g, 16)]
            vav = m4av[pl.ds(16 * g, 16)]
            for k in range(16):
                b = 16 * g + k
                oru = vru[k]
                oau = vau[k]
                orv = vrv[k]
                oav = vav[k]
                u0 = bru[b, pl.ds(oru, L)] + bau[b, pl.ds(oau, L)]
                u1 = bru[b, pl.ds(oru + L, L)] + bau[b, pl.ds(oau + L, L)]
                v0 = brv[b, pl.ds(orv, L)] + bav[b, pl.ds(oav, L)]
                v1 = brv[b, pl.ds(orv + L, L)] + bav[b, pl.ds(oav + L, L)]
                uu[b, pl.ds(0, L)] = u0
                uu[b, pl.ds(L, L)] = u1
                s0 = s0 + u0 * v0
                s1 = s1 + u1 * v1
            return s0, s1

        z = jnp.zeros((L,), jnp.float32)
        s0, s1 = lax.fori_loop(0, bpw // 16, pos_body, (z, z))

        n0, n1 = z, z
        for c in range(NCHUNK):
            stage(rnpos_h, nbase + ncs * c, ncs, i4n1, m4n1)
            stage(anpos_h, nbase + ncs * c, ncs, i4n2, m4n2)
            e0 = pltpu.async_copy(rvw_h.at[i4n1], bn1, sem)
            e1 = pltpu.async_copy(avw_h.at[i4n2], bn2, sem)
            e0.wait(); e1.wait()

            def neg_body(gl, carry):
                m0, m1 = carry
                v1m = m4n1[pl.ds(16 * gl, 16)]
                v2m = m4n2[pl.ds(16 * gl, 16)]
                for k in range(16):
                    j = 16 * gl + k
                    b = bpc * c + j // NEG
                    o1 = v1m[k]
                    o2 = v2m[k]
                    nv0 = bn1[j, pl.ds(o1, L)] + bn2[j, pl.ds(o2, L)]
                    nv1 = bn1[j, pl.ds(o1 + L, L)] + bn2[j, pl.ds(o2 + L, L)]
                    u0 = uu[b, pl.ds(0, L)]
                    u1 = uu[b, pl.ds(L, L)]
                    m0 = m0 + u0 * nv0
                    m1 = m1 + u1 * nv1
                return m0, m1

            n0, n1 = lax.fori_loop(0, ncs // 16, neg_body, (n0, n1))

        for k in range(8):
            ostage[pl.ds(16 * k, 16)] = z
        ostage[pl.ds(0, L)] = s0 + s1
        ostage[pl.ds(L, L)] = n0 + n1
        pltpu.sync_copy(ostage, out_h.at[wid])

    return sc_kernel, L


def kernel(upos, vpos, npos, ru_weight, rv_weight, au_weight, av_weight):
    B = upos.shape[1]
    V = ru_weight.shape[0]
    sc_kernel, L = _make_sc_kernel(B)

    rupos, aupos = upos[0], upos[2]
    rvpos, avpos = vpos[0], vpos[2]
    rnpos = npos[0].reshape(-1)
    anpos = npos[2].reshape(-1)

    ruw4 = ru_weight.reshape(V // 4, 128)
    auw4 = au_weight.reshape(V // 4, 128)
    rvw4 = rv_weight.reshape(V // 4, 128)
    avw4 = av_weight.reshape(V // 4, 128)

    parts = sc_kernel(rupos, aupos, rvpos, avpos, rnpos, anpos,
                      ruw4, auw4, rvw4, avw4)
    score = jnp.sum(parts[:, :L])
    neg_score = jnp.sum(parts[:, L:2 * L])
    return jax.nn.log_sigmoid(score) + jax.nn.log_sigmoid(-neg_score)
